# baseline (speedup 1.0000x reference)
import functools

import jax
import jax.numpy as jnp
from jax import lax
from jax.experimental import pallas as pl
from jax.experimental.pallas import tpu as pltpu
from jax.experimental.pallas import tpu_sc as plsc

N = 10000
E = 320000
D = 128
DP = D // 2

NC = 2
NS = 16
NW = NC * NS
CHUNK = 96
EPW = E // NW
NCHUNK = EPW // CHUNK
TB = EPW - NCHUNK * CHUNK
NPA = 10240
RPT = NPA // NS
MASKHI = -65536


@functools.partial(
    pl.kernel,
    mesh=plsc.VectorSubcoreMesh(core_axis_name="c", subcore_axis_name="s"),
    compiler_params=pltpu.CompilerParams(use_tc_tiling_on_sc=False),
    out_type=jax.ShapeDtypeStruct((NC, NPA, D), jnp.float32),
    scratch_types=[
        pltpu.VMEM((2, CHUNK), jnp.int32),
        pltpu.VMEM((5, CHUNK), jnp.int32),
        pltpu.VMEM((2, CHUNK), jnp.float32),
        pltpu.VMEM((2, CHUNK, DP), jnp.int32),
        pltpu.VMEM((2, CHUNK, D), jnp.float32),
        pltpu.VMEM((TB,), jnp.int32),
        pltpu.VMEM((1, TB), jnp.int32),
        pltpu.VMEM((TB,), jnp.float32),
        pltpu.VMEM((TB, D), jnp.float32),
        pltpu.VMEM_SHARED((NPA, D), jnp.float32),
        pltpu.SemaphoreType.DMA,
        pltpu.SemaphoreType.DMA,
        pltpu.SemaphoreType.DMA,
        pltpu.SemaphoreType.DMA,
        pltpu.SemaphoreType.DMA,
        pltpu.SemaphoreType.DMA,
        pltpu.SemaphoreType.DMA,
        pltpu.SemaphoreType.DMA,
        pltpu.SemaphoreType.DMA,
        pltpu.SemaphoreType.DMA,
    ],
)
def _sc_aggregate(xp_hbm, x_hbm, col_hbm, row_hbm, w_hbm, out_hbm,
                  colb, rowb, wb_, bufp, buff, tcol, trow, tw, tbuf, accum,
                  g0, g1, i0, i1, r0, r1, sc0, sc1, tg, tsc):
    c = lax.axis_index("c")
    s = lax.axis_index("s")
    wid = c * NS + s
    ebase = wid * EPW
    gsem = (g0, g1)
    isem = (i0, i1)
    rsem = (r0, r1)
    ssem = (sc0, sc1)

    def _zero_body(i, carry):
        for b in range(2):
            for k in range(D // 16):
                buff[b, i, pl.ds(k * 16, 16)] = jnp.zeros((16,), jnp.float32)
        return carry
    lax.fori_loop(0, CHUNK, _zero_body, 0)
    for k in range(CHUNK // 16):
        rowb[4, pl.ds(k * 16, 16)] = jnp.zeros((16,), jnp.int32)
    for t in range(RPT // CHUNK):
        pltpu.sync_copy(buff.at[0],
                        accum.at[pl.ds(s * RPT + t * CHUNK, CHUNK)])
    _REM = RPT - (RPT // CHUNK) * CHUNK
    if _REM:
        pltpu.sync_copy(buff.at[0].at[pl.ds(0, _REM)],
                        accum.at[pl.ds(s * RPT + RPT - _REM, _REM)])

    pltpu.sync_copy(col_hbm.at[pl.ds(ebase, CHUNK)], colb.at[0])
    pltpu.sync_copy(w_hbm.at[pl.ds(ebase, CHUNK)], wb_.at[0])
    pltpu.async_copy(row_hbm.at[pl.ds(ebase, CHUNK)], rowb.at[0], r0)
    pltpu.async_copy(xp_hbm.at[colb.at[0]], bufp.at[0], g0)
    pltpu.async_copy(col_hbm.at[pl.ds(ebase + CHUNK, CHUNK)], colb.at[1], i1)
    pltpu.async_copy(w_hbm.at[pl.ds(ebase + CHUNK, CHUNK)], wb_.at[1], i1)
    pltpu.async_copy(row_hbm.at[pl.ds(ebase + CHUNK, CHUNK)], rowb.at[1], r1)
    pltpu.async_copy(buff.at[1], accum.at[rowb.at[4]], sc1, add=True)
    tbase = ebase + NCHUNK * CHUNK
    pltpu.sync_copy(col_hbm.at[pl.ds(tbase, TB)], tcol)
    pltpu.sync_copy(row_hbm.at[pl.ds(tbase, TB)], trow.at[0])
    pltpu.sync_copy(w_hbm.at[pl.ds(tbase, TB)], tw)
    pltpu.async_copy(x_hbm.at[tcol], tbuf, tg)
    plsc.subcore_barrier()

    pltpu.make_async_copy(x_hbm.at[tcol], tbuf, tg).wait()
    w16t = tw[pl.ds(0, TB)]
    for i in range(TB):
        wvt = jnp.full((16,), w16t[i], jnp.float32)
        for k in range(D // 16):
            sl = pl.ds(k * 16, 16)
            tbuf[i, sl] = tbuf[i, sl] * wvt
    pltpu.async_copy(tbuf, accum.at[trow.at[0]], tsc, add=True)

    def _pair_body(t, carry):
        for b in range(2):
            o = 1 - b
            j = t * 2 + b

            @pl.when(j + 1 < NCHUNK)
            def _():
                pltpu.make_async_copy(
                    col_hbm.at[pl.ds(ebase, CHUNK)], colb.at[o], isem[o]).wait()
                pltpu.make_async_copy(
                    w_hbm.at[pl.ds(ebase, CHUNK)], wb_.at[o], isem[o]).wait()
                pltpu.make_async_copy(
                    buff.at[o], accum.at[rowb.at[4]], ssem[o]).wait()
                pltpu.async_copy(xp_hbm.at[colb.at[o]], bufp.at[o], gsem[o])

            pltpu.make_async_copy(
                xp_hbm.at[colb.at[b]], bufp.at[b], gsem[b]).wait()

            def _scale_group(g, carry2):
                w16 = wb_[b, pl.ds(g * 16, 16)]
                for i in range(16):
                    wvec = jnp.full((16,), w16[i], jnp.float32)
                    e = g * 16 + i
                    for m in range(D // 32):
                        wd = bufp[b, e, pl.ds(m * 16, 16)]
                        lo = lax.bitcast_convert_type(wd << 16, jnp.float32)
                        hi = lax.bitcast_convert_type(wd & jnp.int32(MASKHI),
                                                      jnp.float32)
                        buff[b, e, pl.ds(m * 32, 16)] = lo * wvec
                        buff[b, e, pl.ds(m * 32 + 16, 16)] = hi * wvec
                return carry2
            lax.fori_loop(0, CHUNK // 16, _scale_group, 0)

            rslot = lax.rem(j, 4)
            pltpu.make_async_copy(
                row_hbm.at[pl.ds(ebase, CHUNK)], rowb.at[rslot], rsem[b]).wait()
            pltpu.async_copy(buff.at[b], accum.at[rowb.at[rslot]], ssem[b],
                             add=True)

            @pl.when(j + 2 < NCHUNK)
            def _():
                nb = ebase + (j + 2) * CHUNK
                pltpu.async_copy(col_hbm.at[pl.ds(nb, CHUNK)], colb.at[b], isem[b])
                pltpu.async_copy(w_hbm.at[pl.ds(nb, CHUNK)], wb_.at[b], isem[b])
                pltpu.async_copy(row_hbm.at[pl.ds(nb, CHUNK)],
                                 rowb.at[lax.rem(j + 2, 4)], rsem[b])
        return carry
    lax.fori_loop(0, NCHUNK // 2, _pair_body, 0)

    pltpu.make_async_copy(buff.at[0], accum.at[rowb.at[4]], sc0).wait()
    pltpu.make_async_copy(buff.at[1], accum.at[rowb.at[4]], sc1).wait()
    pltpu.make_async_copy(tbuf, accum.at[trow.at[0]], tsc).wait()

    plsc.subcore_barrier()
    pltpu.sync_copy(accum.at[pl.ds(s * RPT, RPT)],
                    out_hbm.at[c].at[pl.ds(s * RPT, RPT)])


BN = 1000


def _tc_linear_prelu(p_ref, wt_ref, b_ref, alpha_ref, o_ref):
    agg = p_ref[0] + p_ref[1]
    h = jnp.dot(agg, wt_ref[...], preferred_element_type=jnp.float32)
    h = h + b_ref[...]
    o_ref[...] = jnp.where(h >= 0, h, h * alpha_ref[...])


def kernel(x, edge_index, weights, W, b, alpha):
    col = edge_index[1].astype(jnp.int32)
    row = edge_index[0].astype(jnp.int32)
    w = weights.astype(jnp.float32)

    xr = x.reshape(N, D // 32, 2, 16).astype(jnp.bfloat16)
    xpair = jnp.stack([xr[:, :, 0, :], xr[:, :, 1, :]], axis=-1)
    xpack = lax.bitcast_convert_type(xpair, jnp.int32).reshape(N, DP)

    partials = _sc_aggregate(xpack, x, col, row, w)

    wt = W.T
    b2 = b.reshape(1, D)
    alpha2 = alpha.reshape(1, D)
    out = pl.pallas_call(
        _tc_linear_prelu,
        grid=(N // BN,),
        in_specs=[
            pl.BlockSpec((NC, BN, D), lambda i: (0, i, 0)),
            pl.BlockSpec((D, D), lambda i: (0, 0)),
            pl.BlockSpec((1, D), lambda i: (0, 0)),
            pl.BlockSpec((1, D), lambda i: (0, 0)),
        ],
        out_specs=pl.BlockSpec((BN, D), lambda i: (i, 0)),
        out_shape=jax.ShapeDtypeStruct((N, D), jnp.float32),
    )(partials, wt, b2, alpha2)
    return out


# speedup vs baseline: 2.0577x; 2.0577x over previous
import functools

import jax
import jax.numpy as jnp
from jax import lax
from jax.experimental import pallas as pl
from jax.experimental.pallas import tpu as pltpu
from jax.experimental.pallas import tpu_sc as plsc

N = 10000
E = 320000
D = 128

NC = 2
NS = 16
NW = NC * NS
CHUNK = 128
EPW = E // NW
NCHUNK = EPW // CHUNK
TB = EPW - NCHUNK * CHUNK
NPA = 10240
RPT = NPA // NS


@functools.partial(
    pl.kernel,
    mesh=plsc.VectorSubcoreMesh(core_axis_name="c", subcore_axis_name="s"),
    out_type=jax.ShapeDtypeStruct((NC, NPA, D), jnp.float32),
    scratch_types=[
        pltpu.VMEM((2, CHUNK), jnp.int32),
        pltpu.VMEM((5, CHUNK), jnp.int32),
        pltpu.VMEM((2, CHUNK), jnp.float32),
        pltpu.VMEM((2, CHUNK, D), jnp.float32),
        pltpu.VMEM((TB,), jnp.int32),
        pltpu.VMEM((1, TB), jnp.int32),
        pltpu.VMEM((TB,), jnp.float32),
        pltpu.VMEM((TB, D), jnp.float32),
        pltpu.VMEM_SHARED((NPA, D), jnp.float32),
        pltpu.SemaphoreType.DMA,
        pltpu.SemaphoreType.DMA,
        pltpu.SemaphoreType.DMA,
        pltpu.SemaphoreType.DMA,
        pltpu.SemaphoreType.DMA,
        pltpu.SemaphoreType.DMA,
        pltpu.SemaphoreType.DMA,
        pltpu.SemaphoreType.DMA,
        pltpu.SemaphoreType.DMA,
        pltpu.SemaphoreType.DMA,
    ],
)
def _sc_aggregate(x_hbm, col_hbm, row_hbm, w_hbm, out_hbm,
                  colb, rowb, wb_, bufs, tcol, trow, tw, tbuf, accum,
                  g0, g1, i0, i1, r0, r1, sc0, sc1, tg, tsc):
    c = lax.axis_index("c")
    s = lax.axis_index("s")
    wid = c * NS + s
    ebase = wid * EPW
    gsem = (g0, g1)
    isem = (i0, i1)
    rsem = (r0, r1)
    ssem = (sc0, sc1)

    def _zero_body(i, carry):
        for b in range(2):
            for k in range(D // 16):
                bufs[b, i, pl.ds(k * 16, 16)] = jnp.zeros((16,), jnp.float32)
        return carry
    lax.fori_loop(0, CHUNK, _zero_body, 0)
    for k in range(CHUNK // 16):
        rowb[4, pl.ds(k * 16, 16)] = jnp.zeros((16,), jnp.int32)
    for t in range(RPT // CHUNK):
        pltpu.sync_copy(bufs.at[0], accum.at[pl.ds(s * RPT + t * CHUNK, CHUNK)])

    pltpu.sync_copy(col_hbm.at[pl.ds(ebase, CHUNK)], colb.at[0])
    pltpu.sync_copy(w_hbm.at[pl.ds(ebase, CHUNK)], wb_.at[0])
    pltpu.async_copy(row_hbm.at[pl.ds(ebase, CHUNK)], rowb.at[0], r0)
    pltpu.async_copy(x_hbm.at[colb.at[0]], bufs.at[0], g0)
    pltpu.async_copy(col_hbm.at[pl.ds(ebase + CHUNK, CHUNK)], colb.at[1], i1)
    pltpu.async_copy(w_hbm.at[pl.ds(ebase + CHUNK, CHUNK)], wb_.at[1], i1)
    pltpu.async_copy(row_hbm.at[pl.ds(ebase + CHUNK, CHUNK)], rowb.at[1], r1)
    pltpu.async_copy(bufs.at[1], accum.at[rowb.at[4]], sc1, add=True)
    tbase = ebase + NCHUNK * CHUNK
    pltpu.sync_copy(col_hbm.at[pl.ds(tbase, TB)], tcol)
    pltpu.sync_copy(row_hbm.at[pl.ds(tbase, TB)], trow.at[0])
    pltpu.sync_copy(w_hbm.at[pl.ds(tbase, TB)], tw)
    pltpu.async_copy(x_hbm.at[tcol], tbuf, tg)
    plsc.subcore_barrier()

    pltpu.make_async_copy(x_hbm.at[tcol], tbuf, tg).wait()
    w16t = tw[pl.ds(0, TB)]
    for i in range(TB):
        wvt = jnp.full((16,), w16t[i], jnp.float32)
        for k in range(D // 16):
            sl = pl.ds(k * 16, 16)
            tbuf[i, sl] = tbuf[i, sl] * wvt
    pltpu.async_copy(tbuf, accum.at[trow.at[0]], tsc, add=True)

    def _pair_body(t, carry):
        for b in range(2):
            o = 1 - b
            j = t * 2 + b

            @pl.when(j + 1 < NCHUNK)
            def _():
                pltpu.make_async_copy(
                    col_hbm.at[pl.ds(ebase, CHUNK)], colb.at[o], isem[o]).wait()
                pltpu.make_async_copy(
                    w_hbm.at[pl.ds(ebase, CHUNK)], wb_.at[o], isem[o]).wait()
                pltpu.make_async_copy(
                    bufs.at[o], accum.at[rowb.at[4]], ssem[o]).wait()
                pltpu.async_copy(x_hbm.at[colb.at[o]], bufs.at[o], gsem[o])

            pltpu.make_async_copy(
                x_hbm.at[colb.at[b]], bufs.at[b], gsem[b]).wait()

            def _scale_group(g, carry2):
                w16 = wb_[b, pl.ds(g * 16, 16)]
                for i in range(16):
                    wvec = jnp.full((16,), w16[i], jnp.float32)
                    e = g * 16 + i
                    for k in range(D // 16):
                        sl = pl.ds(k * 16, 16)
                        bufs[b, e, sl] = bufs[b, e, sl] * wvec
                return carry2
            lax.fori_loop(0, CHUNK // 16, _scale_group, 0)

            rslot = lax.rem(j, 4)
            pltpu.make_async_copy(
                row_hbm.at[pl.ds(ebase, CHUNK)], rowb.at[rslot], rsem[b]).wait()
            pltpu.async_copy(bufs.at[b], accum.at[rowb.at[rslot]], ssem[b],
                             add=True)

            @pl.when(j + 2 < NCHUNK)
            def _():
                nb = ebase + (j + 2) * CHUNK
                pltpu.async_copy(col_hbm.at[pl.ds(nb, CHUNK)], colb.at[b], isem[b])
                pltpu.async_copy(w_hbm.at[pl.ds(nb, CHUNK)], wb_.at[b], isem[b])
                pltpu.async_copy(row_hbm.at[pl.ds(nb, CHUNK)],
                                 rowb.at[lax.rem(j + 2, 4)], rsem[b])
        return carry
    lax.fori_loop(0, NCHUNK // 2, _pair_body, 0)

    pltpu.make_async_copy(bufs.at[0], accum.at[rowb.at[4]], sc0).wait()
    pltpu.make_async_copy(bufs.at[1], accum.at[rowb.at[4]], sc1).wait()
    pltpu.make_async_copy(tbuf, accum.at[trow.at[0]], tsc).wait()

    plsc.subcore_barrier()
    pltpu.sync_copy(accum.at[pl.ds(s * RPT, RPT)],
                    out_hbm.at[c].at[pl.ds(s * RPT, RPT)])


BN = 1000


def _tc_linear_prelu(p_ref, wt_ref, b_ref, alpha_ref, o_ref):
    agg = p_ref[0] + p_ref[1]
    h = jnp.dot(agg, wt_ref[...], preferred_element_type=jnp.float32)
    h = h + b_ref[...]
    o_ref[...] = jnp.where(h >= 0, h, h * alpha_ref[...])


def kernel(x, edge_index, weights, W, b, alpha):
    col = edge_index[1].astype(jnp.int32)
    row = edge_index[0].astype(jnp.int32)
    w = weights.astype(jnp.float32)

    partials = _sc_aggregate(x, col, row, w)

    wt = W.T
    b2 = b.reshape(1, D)
    alpha2 = alpha.reshape(1, D)
    out = pl.pallas_call(
        _tc_linear_prelu,
        grid=(N // BN,),
        in_specs=[
            pl.BlockSpec((NC, BN, D), lambda i: (0, i, 0)),
            pl.BlockSpec((D, D), lambda i: (0, 0)),
            pl.BlockSpec((1, D), lambda i: (0, 0)),
            pl.BlockSpec((1, D), lambda i: (0, 0)),
        ],
        out_specs=pl.BlockSpec((BN, D), lambda i: (i, 0)),
        out_shape=jax.ShapeDtypeStruct((N, D), jnp.float32),
    )(partials, wt, b2, alpha2)
    return out
